# baseline (speedup 1.0000x reference)
import jax
import jax.numpy as jnp
from jax import lax
from jax.experimental import pallas as pl
from jax.experimental.pallas import tpu as pltpu
from jax.experimental.pallas import tpu_sc as plsc

VOCAB = 100000
D = 128
B = 16384
NEG = 5
SPLITS = 2
BH = B // SPLITS

NC = 2
NS = 16
NW = NC * NS
CHUNK = 128
DEPTH = 6

U_PER_W = BH // NW
N_PER_W = BH * NEG // NW
N_CHUNKS = (2 * U_PER_W + N_PER_W) // CHUNK


def _sc_gather_body(u_hbm, v_hbm, pu_hbm, pv_hbm, nf_hbm,
                    ug_hbm, vg_hbm, ng_hbm,
                    idxu, idxv, idxn,
                    b0, b1, b2, b3, b4, b5,
                    g0, g1, g2, g3, g4, g5,
                    w0, w1, w2, w3, w4, w5):
  bufs = [b0, b1, b2, b3, b4, b5]
  gsem = [g0, g1, g2, g3, g4, g5]
  wsem = [w0, w1, w2, w3, w4, w5]
  c = lax.axis_index("c")
  s = lax.axis_index("s")
  wid = s * NC + c

  h0 = pltpu.async_copy(pu_hbm.at[pl.ds(wid * U_PER_W, U_PER_W)], idxu, wsem[0])
  h1 = pltpu.async_copy(pv_hbm.at[pl.ds(wid * U_PER_W, U_PER_W)], idxv, wsem[1])
  h2 = pltpu.async_copy(nf_hbm.at[pl.ds(wid * N_PER_W, N_PER_W)], idxn, wsem[2])
  h0.wait()
  h1.wait()
  h2.wait()

  chunks = []
  for j in range(U_PER_W // CHUNK):
    chunks.append((u_hbm, idxu, j * CHUNK, ug_hbm, wid * U_PER_W + j * CHUNK))
  for j in range(U_PER_W // CHUNK):
    chunks.append((v_hbm, idxv, j * CHUNK, vg_hbm, wid * U_PER_W + j * CHUNK))
  for j in range(N_PER_W // CHUNK):
    chunks.append((v_hbm, idxn, j * CHUNK, ng_hbm, wid * N_PER_W + j * CHUNK))

  gh = [None] * N_CHUNKS
  wh = [None] * N_CHUNKS

  def start_gather(t):
    tbl, iref, ioff, _, _ = chunks[t]
    b = t % DEPTH
    gh[t] = pltpu.async_copy(tbl.at[iref.at[pl.ds(ioff, CHUNK)]],
                             bufs[b], gsem[b])

  for t in range(min(DEPTH, N_CHUNKS)):
    start_gather(t)
  for t in range(N_CHUNKS):
    b = t % DEPTH
    gh[t].wait()
    _, _, _, out_hbm, ooff = chunks[t]
    wh[t] = pltpu.async_copy(bufs[b], out_hbm.at[pl.ds(ooff, CHUNK)], wsem[b])
    if t + DEPTH < N_CHUNKS:
      wh[t].wait()
      start_gather(t + DEPTH)
  for t in range(max(0, N_CHUNKS - DEPTH), N_CHUNKS):
    wh[t].wait()


def _sc_gather(u_table, v_table, pos_u, pos_v, neg_flat):
  mesh = plsc.VectorSubcoreMesh(core_axis_name="c", subcore_axis_name="s")
  out_type = [
      jax.ShapeDtypeStruct((BH, D), jnp.float32),
      jax.ShapeDtypeStruct((BH, D), jnp.float32),
      jax.ShapeDtypeStruct((BH * NEG, D), jnp.float32),
  ]
  k = pl.kernel(
      _sc_gather_body,
      out_type=out_type,
      mesh=mesh,
      scratch_types=(
          [pltpu.VMEM((U_PER_W,), jnp.int32),
           pltpu.VMEM((U_PER_W,), jnp.int32),
           pltpu.VMEM((N_PER_W,), jnp.int32)]
          + [pltpu.VMEM((CHUNK, D), jnp.float32) for _ in range(DEPTH)]
          + [pltpu.SemaphoreType.DMA for _ in range(2 * DEPTH)]
      ),
  )
  return k(u_table, v_table, pos_u, pos_v, neg_flat)


CB = 512
NBLK = BH // CB

_TWO_PI = 6.283185307179586
_PI = 3.141592653589793
_HALF_PI = 1.5707963267948966
_S3 = -0.16666666666666666
_S5 = 0.008333333333333333
_S7 = -0.0001984126984126984
_S9 = 2.7557319223985893e-06


def _sin_poly(x):
  r = lax.rem(x, jnp.float32(_TWO_PI))
  r = jnp.where(r > _PI, r - _TWO_PI, r)
  r = jnp.where(r < -_PI, r + _TWO_PI, r)
  r = jnp.where(r > _HALF_PI, _PI - r, r)
  r = jnp.where(r < -_HALF_PI, -_PI - r, r)
  r2 = r * r
  p = jnp.float32(_S9)
  p = p * r2 + _S7
  p = p * r2 + _S5
  p = p * r2 + _S3
  p = p * r2 + 1.0
  return p * r


def _tc_loss_body(t_ref, f_ref, ug_ref, vg_ref, ng_ref, o_ref):
  i = pl.program_id(0)
  te = _sin_poly(t_ref[...] * f_ref[...])
  eu = ug_ref[...] + te
  ones = jnp.ones((D, 1), jnp.float32)
  s = jnp.clip(jnp.dot(eu * vg_ref[...], ones,
                       preferred_element_type=jnp.float32), -10.0, 10.0)
  acc = jnp.sum(jnp.log1p(jnp.exp(-s)))
  eur = jnp.broadcast_to(eu[:, None, :], (CB, NEG, D)).reshape(CB * NEG, D)
  ns = jnp.clip(jnp.dot(ng_ref[...] * eur, ones,
                        preferred_element_type=jnp.float32), -10.0, 10.0)
  acc = acc + jnp.sum(jnp.log1p(jnp.exp(ns)))

  @pl.when(i == 0)
  def _():
    o_ref[0, 0] = 0.0

  o_ref[0, 0] += acc


def _tc_loss(time_h, freq_emb, ug, vg, ng):
  t2 = time_h.reshape(BH, 1)
  f2 = freq_emb.reshape(1, D)
  in_specs = [
      pl.BlockSpec((CB, 1), lambda i: (i, 0)),
      pl.BlockSpec((1, D), lambda i: (0, 0)),
      pl.BlockSpec((CB, D), lambda i: (i, 0)),
      pl.BlockSpec((CB, D), lambda i: (i, 0)),
      pl.BlockSpec((CB * NEG, D), lambda i: (i, 0)),
  ]
  out = pl.pallas_call(
      _tc_loss_body,
      grid=(NBLK,),
      in_specs=in_specs,
      out_specs=pl.BlockSpec((1, 1), lambda i: (0, 0),
                             memory_space=pltpu.SMEM),
      out_shape=jax.ShapeDtypeStruct((1, 1), jnp.float32),
  )(t2, f2, ug, vg, ng)
  return out


def kernel(u_table, v_table, freq_emb, time, pos_u, pos_v, neg_v):
  pu = pos_u.astype(jnp.int32)
  pv = pos_v.astype(jnp.int32)
  nf_all = neg_v.astype(jnp.int32).reshape(-1)

  gathered = []
  for h in range(SPLITS):
    lo = h * BH
    gathered.append(
        _sc_gather(u_table, v_table, pu[lo:lo + BH], pv[lo:lo + BH],
                   nf_all[lo * NEG:(lo + BH) * NEG]))

  acc = jnp.float32(0.0)
  for h in range(SPLITS):
    ug, vg, ng = gathered[h]
    acc = acc + _tc_loss(time[h * BH:(h + 1) * BH], freq_emb, ug, vg, ng)[0, 0]
  return acc / B


# speedup vs baseline: 1.3875x; 1.3875x over previous
import jax
import jax.numpy as jnp
from jax import lax
from jax.experimental import pallas as pl
from jax.experimental.pallas import tpu as pltpu
from jax.experimental.pallas import tpu_sc as plsc

VOCAB = 100000
D = 128
B = 16384
NEG = 5
HALVES = 2
BH = B // HALVES

NC = 2
NS = 16
NW = NC * NS
CHUNK = 128
DEPTH = 6

U_PER_W = BH // NW
N_PER_W = BH * NEG // NW
N_CHUNKS = (2 * U_PER_W + N_PER_W) // CHUNK


def _sc_gather_body(u_hbm, v_hbm, pu_hbm, pv_hbm, nf_hbm,
                    ug_hbm, vg_hbm, ng_hbm,
                    idxu, idxv, idxn,
                    b0, b1, b2, b3, b4, b5,
                    g0, g1, g2, g3, g4, g5,
                    w0, w1, w2, w3, w4, w5):
  bufs = [b0, b1, b2, b3, b4, b5]
  gsem = [g0, g1, g2, g3, g4, g5]
  wsem = [w0, w1, w2, w3, w4, w5]
  c = lax.axis_index("c")
  s = lax.axis_index("s")
  wid = s * NC + c

  h0 = pltpu.async_copy(pu_hbm.at[pl.ds(wid * U_PER_W, U_PER_W)], idxu, wsem[0])
  h1 = pltpu.async_copy(pv_hbm.at[pl.ds(wid * U_PER_W, U_PER_W)], idxv, wsem[1])
  h2 = pltpu.async_copy(nf_hbm.at[pl.ds(wid * N_PER_W, N_PER_W)], idxn, wsem[2])
  h0.wait()
  h1.wait()
  h2.wait()

  chunks = []
  for j in range(U_PER_W // CHUNK):
    chunks.append((u_hbm, idxu, j * CHUNK, ug_hbm, wid * U_PER_W + j * CHUNK))
  for j in range(U_PER_W // CHUNK):
    chunks.append((v_hbm, idxv, j * CHUNK, vg_hbm, wid * U_PER_W + j * CHUNK))
  for j in range(N_PER_W // CHUNK):
    chunks.append((v_hbm, idxn, j * CHUNK, ng_hbm, wid * N_PER_W + j * CHUNK))

  gh = [None] * N_CHUNKS
  wh = [None] * N_CHUNKS

  def start_gather(t):
    tbl, iref, ioff, _, _ = chunks[t]
    b = t % DEPTH
    gh[t] = pltpu.async_copy(tbl.at[iref.at[pl.ds(ioff, CHUNK)]],
                             bufs[b], gsem[b])

  for t in range(DEPTH):
    start_gather(t)
  for t in range(N_CHUNKS):
    b = t % DEPTH
    gh[t].wait()
    _, _, _, out_hbm, ooff = chunks[t]
    wh[t] = pltpu.async_copy(bufs[b], out_hbm.at[pl.ds(ooff, CHUNK)], wsem[b])
    if t + DEPTH < N_CHUNKS:
      wh[t].wait()
      start_gather(t + DEPTH)
  for t in range(N_CHUNKS - DEPTH, N_CHUNKS):
    wh[t].wait()


def _sc_gather(u_table, v_table, pos_u, pos_v, neg_flat):
  mesh = plsc.VectorSubcoreMesh(core_axis_name="c", subcore_axis_name="s")
  out_type = [
      jax.ShapeDtypeStruct((BH, D), jnp.float32),
      jax.ShapeDtypeStruct((BH, D), jnp.float32),
      jax.ShapeDtypeStruct((BH * NEG, D), jnp.float32),
  ]
  k = pl.kernel(
      _sc_gather_body,
      out_type=out_type,
      mesh=mesh,
      scratch_types=(
          [pltpu.VMEM((U_PER_W,), jnp.int32),
           pltpu.VMEM((U_PER_W,), jnp.int32),
           pltpu.VMEM((N_PER_W,), jnp.int32)]
          + [pltpu.VMEM((CHUNK, D), jnp.float32) for _ in range(DEPTH)]
          + [pltpu.SemaphoreType.DMA for _ in range(2 * DEPTH)]
      ),
  )
  return k(u_table, v_table, pos_u, pos_v, neg_flat)


CB = 512
NBLK = BH // CB

_TWO_PI = 6.283185307179586
_INV_TWO_PI = 0.15915494309189535
_C1 = 0.9999999959723427
_C3 = -0.1666666504335272
_C5 = 0.00833331450987615
_C7 = -0.0001984031108513311
_C9 = 2.7532292011112062e-06
_C11 = -2.470160974620958e-08
_C13 = 1.3533267883357363e-10


def _sin_poly(x):
  n = jnp.round(x * jnp.float32(_INV_TWO_PI))
  r = x - jnp.float32(_TWO_PI) * n
  r2 = r * r
  p = jnp.float32(_C13)
  p = p * r2 + _C11
  p = p * r2 + _C9
  p = p * r2 + _C7
  p = p * r2 + _C5
  p = p * r2 + _C3
  p = p * r2 + _C1
  return p * r


def _tc_loss_body(t_ref, f_ref, ug_ref, vg_ref, n0, n1, n2, n3, n4, o_ref):
  i = pl.program_id(0)
  te = _sin_poly(t_ref[...] * f_ref[...])
  eu = ug_ref[...] + te
  ones = jnp.ones((D, 1), jnp.float32)
  s = jnp.clip(jnp.dot(eu * vg_ref[...], ones,
                       preferred_element_type=jnp.float32), -10.0, 10.0)
  acc = jnp.sum(jnp.log1p(jnp.exp(-s)))
  for nref in (n0, n1, n2, n3, n4):
    ns = jnp.clip(jnp.dot(nref[...] * eu, ones,
                          preferred_element_type=jnp.float32), -10.0, 10.0)
    acc = acc + jnp.sum(jnp.log1p(jnp.exp(ns)))

  @pl.when(i == 0)
  def _():
    o_ref[0, 0] = 0.0

  o_ref[0, 0] += acc


def _tc_loss(time_h, freq_emb, ug, vg, ng):
  t2 = time_h.reshape(BH, 1)
  f2 = freq_emb.reshape(1, D)
  in_specs = [
      pl.BlockSpec((CB, 1), lambda i: (i, 0)),
      pl.BlockSpec((1, D), lambda i: (0, 0)),
      pl.BlockSpec((CB, D), lambda i: (i, 0)),
      pl.BlockSpec((CB, D), lambda i: (i, 0)),
  ] + [
      pl.BlockSpec((CB, D), lambda i, k=k: (k * NBLK + i, 0))
      for k in range(NEG)
  ]
  out = pl.pallas_call(
      _tc_loss_body,
      grid=(NBLK,),
      in_specs=in_specs,
      out_specs=pl.BlockSpec((1, 1), lambda i: (0, 0),
                             memory_space=pltpu.SMEM),
      out_shape=jax.ShapeDtypeStruct((1, 1), jnp.float32),
  )(t2, f2, ug, vg, ng, ng, ng, ng, ng)
  return out


def kernel(u_table, v_table, freq_emb, time, pos_u, pos_v, neg_v):
  pu = pos_u.astype(jnp.int32)
  pv = pos_v.astype(jnp.int32)
  nvi = neg_v.astype(jnp.int32)

  gathered = []
  for h in range(HALVES):
    lo = h * BH
    nf = nvi[lo:lo + BH].T.reshape(-1)
    gathered.append(
        _sc_gather(u_table, v_table, pu[lo:lo + BH], pv[lo:lo + BH], nf))

  acc = jnp.float32(0.0)
  for h in range(HALVES):
    ug, vg, ng = gathered[h]
    acc = acc + _tc_loss(time[h * BH:(h + 1) * BH], freq_emb, ug, vg, ng)[0, 0]
  return acc / B


# speedup vs baseline: 1.4791x; 1.0660x over previous
import jax
import jax.numpy as jnp
from jax import lax
from jax.experimental import pallas as pl
from jax.experimental.pallas import tpu as pltpu
from jax.experimental.pallas import tpu_sc as plsc

VOCAB = 100000
D = 128
B = 16384
NEG = 5
HALVES = 2
BH = B // HALVES

NC = 2
NS = 16
NW = NC * NS
CHUNK = 128
DEPTH = 6

U_PER_W = BH // NW
N_PER_W = BH * NEG // NW
N_CHUNKS = (2 * U_PER_W + N_PER_W) // CHUNK


def _sc_gather_body(u_hbm, v_hbm, pu_hbm, pv_hbm, nf_hbm,
                    ug_hbm, vg_hbm, ng_hbm,
                    idxu, idxv, idxn,
                    b0, b1, b2, b3, b4, b5,
                    g0, g1, g2, g3, g4, g5,
                    w0, w1, w2, w3, w4, w5):
  bufs = [b0, b1, b2, b3, b4, b5]
  gsem = [g0, g1, g2, g3, g4, g5]
  wsem = [w0, w1, w2, w3, w4, w5]
  c = lax.axis_index("c")
  s = lax.axis_index("s")
  wid = s * NC + c

  h0 = pltpu.async_copy(pu_hbm.at[pl.ds(wid * U_PER_W, U_PER_W)], idxu, wsem[0])
  h1 = pltpu.async_copy(pv_hbm.at[pl.ds(wid * U_PER_W, U_PER_W)], idxv, wsem[1])
  h2 = pltpu.async_copy(nf_hbm.at[pl.ds(wid * N_PER_W, N_PER_W)], idxn, wsem[2])
  h0.wait()
  h1.wait()
  h2.wait()

  chunks = []
  for j in range(U_PER_W // CHUNK):
    chunks.append((u_hbm, idxu, j * CHUNK, ug_hbm, wid * U_PER_W + j * CHUNK))
  for j in range(U_PER_W // CHUNK):
    chunks.append((v_hbm, idxv, j * CHUNK, vg_hbm, wid * U_PER_W + j * CHUNK))
  for j in range(N_PER_W // CHUNK):
    chunks.append((v_hbm, idxn, j * CHUNK, ng_hbm, wid * N_PER_W + j * CHUNK))

  gh = [None] * N_CHUNKS
  wh = [None] * N_CHUNKS

  def start_gather(t):
    tbl, iref, ioff, _, _ = chunks[t]
    b = t % DEPTH
    gh[t] = pltpu.async_copy(tbl.at[iref.at[pl.ds(ioff, CHUNK)]],
                             bufs[b], gsem[b])

  for t in range(DEPTH):
    start_gather(t)
  for t in range(N_CHUNKS):
    b = t % DEPTH
    gh[t].wait()
    _, _, _, out_hbm, ooff = chunks[t]
    wh[t] = pltpu.async_copy(bufs[b], out_hbm.at[pl.ds(ooff, CHUNK)], wsem[b])
    if t + DEPTH < N_CHUNKS:
      wh[t].wait()
      start_gather(t + DEPTH)
  for t in range(N_CHUNKS - DEPTH, N_CHUNKS):
    wh[t].wait()


def _sc_gather(u_table, v_table, pos_u, pos_v, neg_flat):
  mesh = plsc.VectorSubcoreMesh(core_axis_name="c", subcore_axis_name="s")
  out_type = [
      jax.ShapeDtypeStruct((BH, D), jnp.float32),
      jax.ShapeDtypeStruct((BH, D), jnp.float32),
      jax.ShapeDtypeStruct((BH * NEG, D), jnp.float32),
  ]
  k = pl.kernel(
      _sc_gather_body,
      out_type=out_type,
      mesh=mesh,
      scratch_types=(
          [pltpu.VMEM((U_PER_W,), jnp.int32),
           pltpu.VMEM((U_PER_W,), jnp.int32),
           pltpu.VMEM((N_PER_W,), jnp.int32)]
          + [pltpu.VMEM((CHUNK, D), jnp.float32) for _ in range(DEPTH)]
          + [pltpu.SemaphoreType.DMA for _ in range(2 * DEPTH)]
      ),
  )
  return k(u_table, v_table, pos_u, pos_v, neg_flat)


CB = 1024
NBLK = BH // CB

_TWO_PI = 6.283185307179586
_INV_TWO_PI = 0.15915494309189535
_C1 = 0.9999999959723427
_C3 = -0.1666666504335272
_C5 = 0.00833331450987615
_C7 = -0.0001984031108513311
_C9 = 2.7532292011112062e-06
_C11 = -2.470160974620958e-08
_C13 = 1.3533267883357363e-10


def _sin_poly(x):
  n = jnp.round(x * jnp.float32(_INV_TWO_PI))
  r = x - jnp.float32(_TWO_PI) * n
  r2 = r * r
  p = jnp.float32(_C13)
  p = p * r2 + _C11
  p = p * r2 + _C9
  p = p * r2 + _C7
  p = p * r2 + _C5
  p = p * r2 + _C3
  p = p * r2 + _C1
  return p * r


def _tc_loss_body(t_ref, f_ref, ug_ref, vg_ref, n0, n1, n2, n3, n4, o_ref):
  i = pl.program_id(0)
  te = _sin_poly(t_ref[...] * f_ref[...])
  eu = ug_ref[...] + te
  ones = jnp.ones((D, 1), jnp.float32)
  s = jnp.clip(jnp.dot(eu * vg_ref[...], ones,
                       preferred_element_type=jnp.float32), -10.0, 10.0)
  acc = jnp.sum(jnp.log1p(jnp.exp(-s)))
  for nref in (n0, n1, n2, n3, n4):
    ns = jnp.clip(jnp.dot(nref[...] * eu, ones,
                          preferred_element_type=jnp.float32), -10.0, 10.0)
    acc = acc + jnp.sum(jnp.log1p(jnp.exp(ns)))

  @pl.when(i == 0)
  def _():
    o_ref[0, 0] = 0.0

  o_ref[0, 0] += acc


def _tc_loss(time_h, freq_emb, ug, vg, ng):
  t2 = time_h.reshape(BH, 1)
  f2 = freq_emb.reshape(1, D)
  in_specs = [
      pl.BlockSpec((CB, 1), lambda i: (i, 0)),
      pl.BlockSpec((1, D), lambda i: (0, 0)),
      pl.BlockSpec((CB, D), lambda i: (i, 0)),
      pl.BlockSpec((CB, D), lambda i: (i, 0)),
  ] + [
      pl.BlockSpec((CB, D), lambda i, k=k: (k * NBLK + i, 0))
      for k in range(NEG)
  ]
  out = pl.pallas_call(
      _tc_loss_body,
      grid=(NBLK,),
      in_specs=in_specs,
      out_specs=pl.BlockSpec((1, 1), lambda i: (0, 0),
                             memory_space=pltpu.SMEM),
      out_shape=jax.ShapeDtypeStruct((1, 1), jnp.float32),
  )(t2, f2, ug, vg, ng, ng, ng, ng, ng)
  return out


def kernel(u_table, v_table, freq_emb, time, pos_u, pos_v, neg_v):
  pu = pos_u.astype(jnp.int32)
  pv = pos_v.astype(jnp.int32)
  nvi = neg_v.astype(jnp.int32)

  gathered = []
  for h in range(HALVES):
    lo = h * BH
    nf = nvi[lo:lo + BH].T.reshape(-1)
    gathered.append(
        _sc_gather(u_table, v_table, pu[lo:lo + BH], pv[lo:lo + BH], nf))

  acc = jnp.float32(0.0)
  for h in range(HALVES):
    ug, vg, ng = gathered[h]
    acc = acc + _tc_loss(time[h * BH:(h + 1) * BH], freq_emb, ug, vg, ng)[0, 0]
  return acc / B
